# baseline (speedup 1.0000x reference)
import functools
import jax
import jax.numpy as jnp
import numpy as np
from jax import lax
from jax.experimental import pallas as pl
from jax.experimental.pallas import tpu as pltpu
from jax.experimental.pallas import tpu_sc as plsc

VOCAB = 1000000
EMBED_DIM = 64
NUM_CLASSES = 128
PAD_IDX = 0
BATCH = 4096
SEQ_LEN = 200

NUM_WORKERS = 32
BPW = BATCH // NUM_WORKERS
SCHUNKS = ((0, 104), (104, 96))
NVREG = EMBED_DIM // 16


def _sc_body(ta_hbm, tb_hbm, table_hbm, out_hbm, idxa_v, idxb_v, rows_v,
             sums_v, sems):
    cid = lax.axis_index("c")
    sid = lax.axis_index("s")
    wid = sid * 2 + cid
    base = wid * BPW
    pltpu.sync_copy(ta_hbm.at[pl.ds(base, BPW)], idxa_v)
    pltpu.sync_copy(tb_hbm.at[pl.ds(base, BPW)], idxb_v)

    def fire(r, buf):
        pltpu.async_copy(
            table_hbm.at[idxa_v.at[r]],
            rows_v.at[buf, pl.ds(0, 128)],
            sems.at[buf],
        )
        pltpu.async_copy(
            table_hbm.at[idxb_v.at[r, pl.ds(0, SEQ_LEN - 128)]],
            rows_v.at[buf, pl.ds(128, SEQ_LEN - 128)],
            sems.at[buf],
        )

    fire(0, 0)

    def row_body(r, _):
        buf = lax.rem(r, 2)

        @pl.when(r + 1 < BPW)
        def _prefetch():
            fire(r + 1, 1 - buf)

        pltpu.make_async_copy(
            table_hbm.at[pl.ds(0, SEQ_LEN)], rows_v.at[buf], sems.at[buf]
        ).wait()

        bc = jax.lax.bitcast_convert_type

        def seq_body(s, accs):
            new = []
            for j in range(2):
                w = bc(rows_v[buf, s, pl.ds(j * 16, 16)], jnp.int32)
                lo = bc(w << 16, jnp.float32)
                hi = bc(w & jnp.int32(-65536), jnp.float32)
                new.append(accs[2 * j] + lo)
                new.append(accs[2 * j + 1] + hi)
            return tuple(new)

        zeros = tuple(jnp.zeros((16,), jnp.float32) for _ in range(NVREG))
        accs = plsc.parallel_loop(0, SEQ_LEN, carry=zeros, unroll=8)(seq_body)
        for j in range(NVREG):
            sums_v[r, pl.ds(j * 16, 16)] = accs[j]
        return 0

    lax.fori_loop(0, BPW, row_body, 0)
    pltpu.sync_copy(sums_v, out_hbm.at[pl.ds(base, BPW), pl.ds(0, EMBED_DIM)])


@functools.partial(jax.jit, static_argnums=())
def _sc_sums(text_a, text_b, emb_table):
    mesh = plsc.VectorSubcoreMesh(core_axis_name="c", subcore_axis_name="s")
    return pl.kernel(
        _sc_body,
        mesh=mesh,
        out_type=jax.ShapeDtypeStruct((BATCH, 2 * EMBED_DIM), jnp.float32),
        scratch_types=[
            pltpu.VMEM((BPW, 128), jnp.int32),
            pltpu.VMEM((BPW, 128), jnp.int32),
            pltpu.VMEM((2, SEQ_LEN, EMBED_DIM // 2), jnp.float32),
            pltpu.VMEM((BPW, EMBED_DIM), jnp.float32),
            pltpu.SemaphoreType.DMA((2,)),
        ],
        compiler_params=pltpu.CompilerParams(use_tc_tiling_on_sc=False),
    )(text_a, text_b, emb_table)


REPACK_C = 4096
REPACK_H = REPACK_C // 2
REPACK_G = -(-VOCAB // REPACK_C)
VOCAB_PAD = REPACK_G * REPACK_C


REPACK_Q = REPACK_C // 4


def _rnd_bf16_bits(x):
    xi = jax.lax.bitcast_convert_type(x, jnp.int32)
    r = xi + jnp.int32(0x7FFF) + ((xi >> 16) & 1)
    return r


def _tc_repack_body(t_ref, out_ref):
    y = jnp.transpose(t_ref[...])
    a = _rnd_bf16_bits(y[:, :32])
    b = _rnd_bf16_bits(y[:, 32:])
    pk_i = (b & jnp.int32(-65536)) | ((a >> 16) & jnp.int32(0xFFFF))
    pk = jax.lax.bitcast_convert_type(pk_i, jnp.float32)
    out_ref[...] = jnp.concatenate(
        [pk[q * REPACK_Q:(q + 1) * REPACK_Q] for q in range(4)], axis=1)


def _tc_repack(t_tbl):
    return pl.pallas_call(
        _tc_repack_body,
        grid=(REPACK_G,),
        in_specs=[pl.BlockSpec((EMBED_DIM, REPACK_C), lambda i: (0, i))],
        out_specs=pl.BlockSpec((REPACK_Q, 2 * EMBED_DIM), lambda i: (i, 0)),
        out_shape=jax.ShapeDtypeStruct((VOCAB_PAD // 4, 2 * EMBED_DIM),
                                       jnp.float32),
    )(t_tbl)


def _remap_idx(i):
    m = i % REPACK_C
    return i - m + 4 * (m % REPACK_Q) + m // REPACK_Q


BB = 256


def _tc_finish_body(text_ref, sums_ref, w_ref, b_ref, out_ref):
    mask = (text_ref[...] != PAD_IDX).astype(jnp.float32)
    cnt = jnp.sum(mask, axis=1, keepdims=True)
    avg = sums_ref[...][:, :EMBED_DIM] / (cnt + 1e-6)
    out_ref[...] = (
        lax.dot_general(
            avg, w_ref[...], (((1,), (1,)), ((), ())),
            preferred_element_type=jnp.float32,
        )
        + b_ref[...]
    )


def _tc_finish(text, sums, fc_w, fc_b2):
    return pl.pallas_call(
        _tc_finish_body,
        grid=(BATCH // BB,),
        in_specs=[
            pl.BlockSpec((BB, SEQ_LEN), lambda i: (i, 0)),
            pl.BlockSpec((BB, 2 * EMBED_DIM), lambda i: (i, 0)),
            pl.BlockSpec((NUM_CLASSES, EMBED_DIM), lambda i: (0, 0)),
            pl.BlockSpec((1, NUM_CLASSES), lambda i: (0, 0)),
        ],
        out_specs=pl.BlockSpec((BB, NUM_CLASSES), lambda i: (i, 0)),
        out_shape=jax.ShapeDtypeStruct((BATCH, NUM_CLASSES), jnp.float32),
    )(text, sums, fc_w, fc_b2)


_PERM = np.concatenate([
    np.arange(0, 16), np.arange(32, 48), np.arange(16, 32), np.arange(48, 64),
])


def kernel(text, emb_table, fc_w, fc_b):
    tbl_lin = _tc_repack(emb_table.T).reshape(VOCAB_PAD, EMBED_DIM // 2)
    text_a = _remap_idx(text[:, :128])
    text_b = _remap_idx(
        jnp.pad(text[:, 128:], ((0, 0), (0, 128 - (SEQ_LEN - 128)))))
    sums = _sc_sums(text_a, text_b, tbl_lin)
    return _tc_finish(text, sums, fc_w[:, _PERM],
                      fc_b.reshape(1, NUM_CLASSES))


# speedup vs baseline: 1.2829x; 1.2829x over previous
import functools
import jax
import jax.numpy as jnp
import numpy as np
from jax import lax
from jax.experimental import pallas as pl
from jax.experimental.pallas import tpu as pltpu
from jax.experimental.pallas import tpu_sc as plsc

VOCAB = 1000000
EMBED_DIM = 64
NUM_CLASSES = 128
PAD_IDX = 0
BATCH = 4096
SEQ_LEN = 200

NUM_WORKERS = 32
BPW = BATCH // NUM_WORKERS
SCHUNKS = ((0, 104), (104, 96))
NVREG = EMBED_DIM // 16


def _sc_body(ta_hbm, tb_hbm, table_hbm, out_hbm, idxa_v, idxb_v, rows_v,
             sums_v, sems):
    cid = lax.axis_index("c")
    sid = lax.axis_index("s")
    wid = sid * 2 + cid
    base = wid * BPW
    pltpu.sync_copy(ta_hbm.at[pl.ds(base, BPW)], idxa_v)
    pltpu.sync_copy(tb_hbm.at[pl.ds(base, BPW)], idxb_v)

    def fire(r, buf):
        pltpu.async_copy(
            table_hbm.at[idxa_v.at[r]],
            rows_v.at[buf, pl.ds(0, 128)],
            sems.at[buf],
        )
        pltpu.async_copy(
            table_hbm.at[idxb_v.at[r, pl.ds(0, SEQ_LEN - 128)]],
            rows_v.at[buf, pl.ds(128, SEQ_LEN - 128)],
            sems.at[buf],
        )

    fire(0, 0)

    def row_body(r, _):
        buf = lax.rem(r, 2)

        @pl.when(r + 1 < BPW)
        def _prefetch():
            fire(r + 1, 1 - buf)

        pltpu.make_async_copy(
            table_hbm.at[pl.ds(0, SEQ_LEN)], rows_v.at[buf], sems.at[buf]
        ).wait()

        def seq_body(s, accs):
            return tuple(
                accs[j] + rows_v[buf, s, pl.ds(j * 16, 16)]
                for j in range(NVREG)
            )

        zeros = tuple(jnp.zeros((16,), jnp.float32) for _ in range(NVREG))
        accs = plsc.parallel_loop(0, SEQ_LEN, carry=zeros, unroll=8)(seq_body)
        for j in range(NVREG):
            sums_v[r, pl.ds(j * 16, 16)] = accs[j]
        return 0

    lax.fori_loop(0, BPW, row_body, 0)
    pltpu.sync_copy(sums_v, out_hbm.at[pl.ds(base, BPW), pl.ds(0, EMBED_DIM)])


@functools.partial(jax.jit, static_argnums=())
def _sc_sums(text_a, text_b, emb_table):
    mesh = plsc.VectorSubcoreMesh(core_axis_name="c", subcore_axis_name="s")
    return pl.kernel(
        _sc_body,
        mesh=mesh,
        out_type=jax.ShapeDtypeStruct((BATCH, 2 * EMBED_DIM), jnp.float32),
        scratch_types=[
            pltpu.VMEM((BPW, 128), jnp.int32),
            pltpu.VMEM((BPW, 128), jnp.int32),
            pltpu.VMEM((2, SEQ_LEN, EMBED_DIM), jnp.float32),
            pltpu.VMEM((BPW, EMBED_DIM), jnp.float32),
            pltpu.SemaphoreType.DMA((2,)),
        ],
        compiler_params=pltpu.CompilerParams(use_tc_tiling_on_sc=False),
    )(text_a, text_b, emb_table)


REPACK_C = 8192
REPACK_H = REPACK_C // 2
REPACK_G = -(-VOCAB // REPACK_C)
VOCAB_PAD = REPACK_G * REPACK_C


def _tc_repack_body(t_ref, out_ref):
    y = jnp.transpose(t_ref[...])
    out_ref[...] = jnp.concatenate([y[:REPACK_H], y[REPACK_H:]], axis=1)


def _tc_repack(t_tbl):
    return pl.pallas_call(
        _tc_repack_body,
        grid=(REPACK_G,),
        in_specs=[pl.BlockSpec((EMBED_DIM, REPACK_C), lambda i: (0, i))],
        out_specs=pl.BlockSpec((REPACK_H, 2 * EMBED_DIM), lambda i: (i, 0)),
        out_shape=jax.ShapeDtypeStruct((VOCAB_PAD // 2, 2 * EMBED_DIM),
                                       jnp.float32),
    )(t_tbl)


def _remap_idx(i):
    m = i % REPACK_C
    return i + m - jnp.where(m >= REPACK_H, REPACK_C - 1, 0)


BB = 256


def _tc_finish_body(text_ref, sums_ref, w_ref, b_ref, out_ref):
    mask = (text_ref[...] != PAD_IDX).astype(jnp.float32)
    cnt = jnp.sum(mask, axis=1, keepdims=True)
    avg = sums_ref[...][:, :EMBED_DIM] / (cnt + 1e-6)
    out_ref[...] = (
        lax.dot_general(
            avg, w_ref[...], (((1,), (1,)), ((), ())),
            preferred_element_type=jnp.float32,
        )
        + b_ref[...]
    )


def _tc_finish(text, sums, fc_w, fc_b2):
    return pl.pallas_call(
        _tc_finish_body,
        grid=(BATCH // BB,),
        in_specs=[
            pl.BlockSpec((BB, SEQ_LEN), lambda i: (i, 0)),
            pl.BlockSpec((BB, 2 * EMBED_DIM), lambda i: (i, 0)),
            pl.BlockSpec((NUM_CLASSES, EMBED_DIM), lambda i: (0, 0)),
            pl.BlockSpec((1, NUM_CLASSES), lambda i: (0, 0)),
        ],
        out_specs=pl.BlockSpec((BB, NUM_CLASSES), lambda i: (i, 0)),
        out_shape=jax.ShapeDtypeStruct((BATCH, NUM_CLASSES), jnp.float32),
    )(text, sums, fc_w, fc_b2)


def kernel(text, emb_table, fc_w, fc_b):
    tbl_lin = _tc_repack(emb_table.T).reshape(VOCAB_PAD, EMBED_DIM)
    text_a = _remap_idx(text[:, :128])
    text_b = _remap_idx(
        jnp.pad(text[:, 128:], ((0, 0), (0, 128 - (SEQ_LEN - 128)))))
    sums = _sc_sums(text_a, text_b, tbl_lin)
    return _tc_finish(text, sums, fc_w, fc_b.reshape(1, NUM_CLASSES))


# speedup vs baseline: 1.3878x; 1.0817x over previous
import functools
import jax
import jax.numpy as jnp
import numpy as np
from jax import lax
from jax.experimental import pallas as pl
from jax.experimental.pallas import tpu as pltpu
from jax.experimental.pallas import tpu_sc as plsc

VOCAB = 1000000
EMBED_DIM = 64
NUM_CLASSES = 128
PAD_IDX = 0
BATCH = 4096
SEQ_LEN = 200

NUM_WORKERS = 32
BPW = BATCH // NUM_WORKERS
SCHUNKS = ((0, 104), (104, 96))
NVREG = EMBED_DIM // 16


def _sc_body(ta_hbm, tb_hbm, table_hbm, out_hbm, idxa_v, idxb_v, rows_v,
             sums_v, sems):
    cid = lax.axis_index("c")
    sid = lax.axis_index("s")
    wid = sid * 2 + cid
    base = wid * BPW
    pltpu.sync_copy(ta_hbm.at[pl.ds(base, BPW)], idxa_v)
    pltpu.sync_copy(tb_hbm.at[pl.ds(base, BPW)], idxb_v)

    def fire(r, buf):
        pltpu.async_copy(
            table_hbm.at[idxa_v.at[r]],
            rows_v.at[buf, pl.ds(0, 128)],
            sems.at[buf],
        )
        pltpu.async_copy(
            table_hbm.at[idxb_v.at[r, pl.ds(0, SEQ_LEN - 128)]],
            rows_v.at[buf, pl.ds(128, SEQ_LEN - 128)],
            sems.at[buf],
        )

    fire(0, 0)

    def row_body(r, _):
        buf = lax.rem(r, 2)

        @pl.when(r + 1 < BPW)
        def _prefetch():
            fire(r + 1, 1 - buf)

        pltpu.make_async_copy(
            table_hbm.at[pl.ds(0, SEQ_LEN)], rows_v.at[buf], sems.at[buf]
        ).wait()

        def seq_body(s, accs):
            return tuple(
                accs[j] + rows_v[buf, s, pl.ds(j * 16, 16)]
                for j in range(NVREG)
            )

        zeros = tuple(jnp.zeros((16,), jnp.float32) for _ in range(NVREG))
        accs = plsc.parallel_loop(0, SEQ_LEN, carry=zeros, unroll=8)(seq_body)
        for j in range(NVREG):
            sums_v[r, pl.ds(j * 16, 16)] = accs[j]
        return 0

    lax.fori_loop(0, BPW, row_body, 0)
    pltpu.sync_copy(sums_v, out_hbm.at[pl.ds(base, BPW), pl.ds(0, EMBED_DIM)])


@functools.partial(jax.jit, static_argnums=())
def _sc_sums(text_a, text_b, emb_table):
    mesh = plsc.VectorSubcoreMesh(core_axis_name="c", subcore_axis_name="s")
    return pl.kernel(
        _sc_body,
        mesh=mesh,
        out_type=jax.ShapeDtypeStruct((BATCH, 2 * EMBED_DIM), jnp.float32),
        scratch_types=[
            pltpu.VMEM((BPW, 128), jnp.int32),
            pltpu.VMEM((BPW, 128), jnp.int32),
            pltpu.VMEM((2, SEQ_LEN, EMBED_DIM), jnp.float32),
            pltpu.VMEM((BPW, EMBED_DIM), jnp.float32),
            pltpu.SemaphoreType.DMA((2,)),
        ],
        compiler_params=pltpu.CompilerParams(use_tc_tiling_on_sc=False),
    )(text_a, text_b, emb_table)


REPACK_C = 16384
REPACK_H = REPACK_C // 2
REPACK_G = -(-VOCAB // REPACK_C)
VOCAB_PAD = REPACK_G * REPACK_C


def _tc_repack_body(t_ref, out_ref):
    y = jnp.transpose(t_ref[...])
    out_ref[...] = jnp.concatenate([y[:REPACK_H], y[REPACK_H:]], axis=1)


def _tc_repack(t_tbl):
    return pl.pallas_call(
        _tc_repack_body,
        grid=(REPACK_G,),
        in_specs=[pl.BlockSpec((EMBED_DIM, REPACK_C), lambda i: (0, i))],
        out_specs=pl.BlockSpec((REPACK_H, 2 * EMBED_DIM), lambda i: (i, 0)),
        out_shape=jax.ShapeDtypeStruct((VOCAB_PAD // 2, 2 * EMBED_DIM),
                                       jnp.float32),
    )(t_tbl)


def _remap_idx(i):
    m = i % REPACK_C
    return i + m - jnp.where(m >= REPACK_H, REPACK_C - 1, 0)


BB = 256


def _tc_finish_body(text_ref, sums_ref, w_ref, b_ref, out_ref):
    mask = (text_ref[...] != PAD_IDX).astype(jnp.float32)
    cnt = jnp.sum(mask, axis=1, keepdims=True)
    avg = sums_ref[...][:, :EMBED_DIM] / (cnt + 1e-6)
    out_ref[...] = (
        lax.dot_general(
            avg, w_ref[...], (((1,), (1,)), ((), ())),
            preferred_element_type=jnp.float32,
        )
        + b_ref[...]
    )


def _tc_finish(text, sums, fc_w, fc_b2):
    return pl.pallas_call(
        _tc_finish_body,
        grid=(BATCH // BB,),
        in_specs=[
            pl.BlockSpec((BB, SEQ_LEN), lambda i: (i, 0)),
            pl.BlockSpec((BB, 2 * EMBED_DIM), lambda i: (i, 0)),
            pl.BlockSpec((NUM_CLASSES, EMBED_DIM), lambda i: (0, 0)),
            pl.BlockSpec((1, NUM_CLASSES), lambda i: (0, 0)),
        ],
        out_specs=pl.BlockSpec((BB, NUM_CLASSES), lambda i: (i, 0)),
        out_shape=jax.ShapeDtypeStruct((BATCH, NUM_CLASSES), jnp.float32),
    )(text, sums, fc_w, fc_b2)


def kernel(text, emb_table, fc_w, fc_b):
    tbl_lin = _tc_repack(emb_table.T).reshape(VOCAB_PAD, EMBED_DIM)
    text_a = _remap_idx(text[:, :128])
    text_b = _remap_idx(
        jnp.pad(text[:, 128:], ((0, 0), (0, 128 - (SEQ_LEN - 128)))))
    sums = _sc_sums(text_a, text_b, tbl_lin)
    return _tc_finish(text, sums, fc_w, fc_b.reshape(1, NUM_CLASSES))
